# Initial kernel scaffold; baseline (speedup 1.0000x reference)
import jax
import jax.numpy as jnp
from jax.experimental import pallas as pl


def kernel(x, h, emb, conv_params, res_params, gru_params):
    raise NotImplementedError("write your pallas kernel here")


# speedup vs baseline: 13.3716x; 13.3716x over previous
import jax
import jax.numpy as jnp
from jax.experimental import pallas as pl
from jax.experimental.pallas import tpu as pltpu

B = 4
L = 2048
E = 64
H = 512
VOCAB = 512
S = 4
HW = 2100
HWP = 2176
RHP = 512
N_RES = 4
EPS = 1e-05
Lp = L // S
KW = 8
T = 512
NT = L // T
TB = 16
NTB = Lp // TB
G3 = 3 * H

_f32 = jnp.float32


def _conv_kernel(xw_ref, emb_ref, w_ref, b_ref, out_ref):
    idx = xw_ref[0]
    oh = (idx == jax.lax.broadcasted_iota(jnp.int32, (T + KW, VOCAB), 1))
    xe = jnp.dot(oh.astype(_f32), emb_ref[:],
                 preferred_element_type=_f32)
    xwin = jnp.concatenate(
        [xe[KW - d:KW - d + T] for d in range(KW)], axis=1)
    y = jnp.dot(xwin, w_ref[:], preferred_element_type=_f32) + b_ref[:]
    y = jnp.maximum(y, 0.0)
    out_ref[0] = y.reshape(T // S, S, HWP).max(axis=1)


def _res_kernel(y_ref, w1_ref, b1_ref, g_ref, bt_ref, w2_ref, b2_ref, out_ref):
    y = y_ref[:]
    for i in range(N_RES):
        r = jnp.maximum(y, 0.0)
        r = jnp.dot(r, w1_ref[i], preferred_element_type=_f32) + b1_ref[i]
        r = jnp.maximum(r, 0.0)
        r = r * g_ref[i] + bt_ref[i]
        y = y + jnp.dot(r, w2_ref[i], preferred_element_type=_f32) + b2_ref[i]
    out_ref[:] = y


def _proj_kernel(y_ref, w_ref, b_ref, out_ref):
    out_ref[:] = (jnp.dot(y_ref[:], w_ref[:], preferred_element_type=_f32)
                  + b_ref[:])


def _gru_kernel(hs0_ref, gif_ref, gib_ref, whh_ref, bhh_ref,
                outf_ref, outb_ref, hs):
    @pl.when(pl.program_id(0) == 0)
    def _():
        hs[:] = hs0_ref[:]

    h = hs[:]
    hf = h[0:B]
    hb = h[B:2 * B]
    for i in range(TB):
        hcat = jnp.concatenate([hf, hb], axis=0)
        g = jnp.dot(hcat, whh_ref[:], preferred_element_type=_f32) + bhh_ref[:]
        ghf = g[0:B, 0:G3]
        ghb = g[B:2 * B, G3:2 * G3]
        gif = gif_ref[:, i, :]
        gib = gib_ref[:, TB - 1 - i, :]

        def gates(gi, gh, hprev):
            rg = jax.nn.sigmoid(gi[:, 0:H] + gh[:, 0:H])
            zg = jax.nn.sigmoid(gi[:, H:2 * H] + gh[:, H:2 * H])
            ng = jnp.tanh(gi[:, 2 * H:3 * H] + rg * gh[:, 2 * H:3 * H])
            return (1.0 - zg) * ng + zg * hprev

        hf = gates(gif, ghf, hf)
        hb = gates(gib, ghb, hb)
        outf_ref[:, i, :] = hf
        outb_ref[:, TB - 1 - i, :] = hb
    hs[:] = jnp.concatenate([hf, hb], axis=0)


def kernel(x, h, emb, conv_params, res_params, gru_params):
    Wc = jnp.zeros((KW * E, HWP), _f32)
    bc = jnp.zeros((1, HWP), _f32)
    off = 0
    for i, (W, b) in enumerate(conv_params):
        nf = W.shape[0]
        for d in range(i + 1):
            Wc = Wc.at[d * E:(d + 1) * E, off:off + nf].set(W[:, 0, i - d, :].T)
        bc = bc.at[0, off:off + nf].set(b)
        off += nf

    xp = jnp.pad(x.astype(jnp.int32), ((0, 0), (KW, 0)), constant_values=VOCAB)
    xw = jnp.stack([xp[:, n * T:n * T + T + KW] for n in range(NT)], axis=1)
    xw = xw.reshape(B * NT, T + KW, 1)

    Yp = pl.pallas_call(
        _conv_kernel,
        grid=(B * NT,),
        in_specs=[
            pl.BlockSpec((1, T + KW, 1), lambda g: (g, 0, 0)),
            pl.BlockSpec((VOCAB, E), lambda g: (0, 0)),
            pl.BlockSpec((KW * E, HWP), lambda g: (0, 0)),
            pl.BlockSpec((1, HWP), lambda g: (0, 0)),
        ],
        out_specs=pl.BlockSpec((1, T // S, HWP), lambda g: (g, 0, 0)),
        out_shape=jax.ShapeDtypeStruct((B * NT, T // S, HWP), _f32),
    )(xw, emb, Wc, bc)
    Yf = Yp.reshape(B * Lp, HWP)

    gm = 1.0 / jnp.sqrt(1.0 + EPS)
    w1 = jnp.stack([jnp.zeros((HWP, RHP), _f32).at[:HW, :400].set(p[0].T)
                    for p in res_params])
    b1 = jnp.stack([jnp.zeros((1, RHP), _f32).at[0, :400].set(p[1])
                    for p in res_params])
    gmul = jnp.stack([jnp.zeros((1, RHP), _f32).at[0, :400].set(p[4] * gm)
                      for p in res_params])
    beta = jnp.stack([jnp.zeros((1, RHP), _f32).at[0, :400].set(p[5])
                      for p in res_params])
    w2 = jnp.stack([jnp.zeros((RHP, HWP), _f32).at[:400, :HW].set(p[2].T)
                    for p in res_params])
    b2 = jnp.stack([jnp.zeros((1, HWP), _f32).at[0, :HW].set(p[3])
                    for p in res_params])

    RM = 256
    Yr = pl.pallas_call(
        _res_kernel,
        grid=(B * Lp // RM,),
        in_specs=[
            pl.BlockSpec((RM, HWP), lambda m: (m, 0)),
            pl.BlockSpec((N_RES, HWP, RHP), lambda m: (0, 0, 0)),
            pl.BlockSpec((N_RES, 1, RHP), lambda m: (0, 0, 0)),
            pl.BlockSpec((N_RES, 1, RHP), lambda m: (0, 0, 0)),
            pl.BlockSpec((N_RES, 1, RHP), lambda m: (0, 0, 0)),
            pl.BlockSpec((N_RES, RHP, HWP), lambda m: (0, 0, 0)),
            pl.BlockSpec((N_RES, 1, HWP), lambda m: (0, 0, 0)),
        ],
        out_specs=pl.BlockSpec((RM, HWP), lambda m: (m, 0)),
        out_shape=jax.ShapeDtypeStruct((B * Lp, HWP), _f32),
    )(Yf, w1, b1, gmul, beta, w2, b2)

    Wih_f, Whh_f, bih_f, bhh_f = gru_params[0]
    Wih_b, Whh_b, bih_b, bhh_b = gru_params[1]
    Wih = jnp.concatenate(
        [jnp.zeros((HWP, G3), _f32).at[:HW, :].set(Wih_f.T),
         jnp.zeros((HWP, G3), _f32).at[:HW, :].set(Wih_b.T)], axis=1)
    bih = jnp.concatenate([bih_f, bih_b])[None, :]

    Gi = pl.pallas_call(
        _proj_kernel,
        grid=(B * Lp // RM,),
        in_specs=[
            pl.BlockSpec((RM, HWP), lambda m: (m, 0)),
            pl.BlockSpec((HWP, 2 * G3), lambda m: (0, 0)),
            pl.BlockSpec((1, 2 * G3), lambda m: (0, 0)),
        ],
        out_specs=pl.BlockSpec((RM, 2 * G3), lambda m: (m, 0)),
        out_shape=jax.ShapeDtypeStruct((B * Lp, 2 * G3), _f32),
    )(Yr, Wih, bih)
    Gi = Gi.reshape(B, Lp, 2 * G3)

    Whh = jnp.concatenate([Whh_f.T, Whh_b.T], axis=1)
    bhh = jnp.concatenate([bhh_f, bhh_b])[None, :]
    hs0 = jnp.concatenate([h[0], h[1]], axis=0)

    ysf, ysb = pl.pallas_call(
        _gru_kernel,
        grid=(NTB,),
        in_specs=[
            pl.BlockSpec((2 * B, H), lambda t: (0, 0)),
            pl.BlockSpec((B, TB, G3), lambda t: (0, t, 0)),
            pl.BlockSpec((B, TB, G3), lambda t: (0, NTB - 1 - t, 1)),
            pl.BlockSpec((H, 2 * G3), lambda t: (0, 0)),
            pl.BlockSpec((1, 2 * G3), lambda t: (0, 0)),
        ],
        out_specs=[
            pl.BlockSpec((B, TB, H), lambda t: (0, t, 0)),
            pl.BlockSpec((B, TB, H), lambda t: (0, NTB - 1 - t, 0)),
        ],
        out_shape=[
            jax.ShapeDtypeStruct((B, Lp, H), _f32),
            jax.ShapeDtypeStruct((B, Lp, H), _f32),
        ],
        scratch_shapes=[pltpu.VMEM((2 * B, H), _f32)],
    )(hs0, Gi, Gi, Whh, bhh)

    out = jnp.concatenate([ysf, ysb], axis=-1)
    hn = jnp.stack([ysf[:, -1, :], ysb[:, 0, :]], axis=0)
    return out, hn
